# Initial kernel scaffold; baseline (speedup 1.0000x reference)
import jax
import jax.numpy as jnp
from jax.experimental import pallas as pl


def kernel(positions, cells, species, cell_shifts, centers, pairs, structure_centers, structure_pairs, structure_offsets):
    raise NotImplementedError("write your pallas kernel here")


# speedup vs baseline: 5.0569x; 5.0569x over previous
import functools
import math

import jax
import jax.numpy as jnp
from jax import lax
from jax.experimental import pallas as pl
from jax.experimental.pallas import tpu as pltpu
from jax.experimental.pallas import tpu_sc as plsc

_PI = math.pi
_A = math.sqrt(4.0 * _PI)
_C1 = _A * math.sqrt(3.0 / (4.0 * _PI))
_C2XY = _A * 0.5 * math.sqrt(15.0 / _PI)
_C2Z2 = _A * 0.25 * math.sqrt(5.0 / _PI)
_C2XX = _A * 0.25 * math.sqrt(15.0 / _PI)
_C30 = _A * 0.25 * math.sqrt(35.0 / (2.0 * _PI))
_C31 = _A * 0.5 * math.sqrt(105.0 / _PI)
_C32 = _A * 0.25 * math.sqrt(21.0 / (2.0 * _PI))
_C33 = _A * 0.25 * math.sqrt(7.0 / _PI)
_C34 = _A * 0.25 * math.sqrt(105.0 / _PI)

_NW = 32
_LANES = 16


def _rsqrt_newton(r2):
    bits = lax.bitcast_convert_type(r2, jnp.int32)
    seed = jnp.int32(0x5F3759DF) - lax.shift_right_logical(bits, 1)
    y = lax.bitcast_convert_type(seed, jnp.float32)
    h = jnp.float32(0.5) * r2
    for _ in range(4):
        y = y * (jnp.float32(1.5) - h * y * y)
    return y


def _sh_components(x, y, z):
    xx = x * x
    yy = y * y
    zz = z * z
    xy = x * y
    yz = y * z
    xz = x * z
    t8 = xx - yy
    t5z = jnp.float32(5.0) * zz - jnp.float32(1.0)
    sh = [None] * 16
    sh[0] = jnp.full((_LANES,), 1.0, jnp.float32)
    sh[1] = jnp.float32(_C1) * y
    sh[2] = jnp.float32(_C1) * z
    sh[3] = jnp.float32(_C1) * x
    sh[4] = jnp.float32(_C2XY) * xy
    sh[5] = jnp.float32(_C2XY) * yz
    sh[6] = jnp.float32(_C2Z2) * (jnp.float32(3.0) * zz - jnp.float32(1.0))
    sh[7] = jnp.float32(_C2XY) * xz
    sh[8] = jnp.float32(_C2XX) * t8
    sh[9] = jnp.float32(_C30) * y * (jnp.float32(3.0) * xx - yy)
    sh[10] = jnp.float32(_C31) * xy * z
    sh[11] = jnp.float32(_C32) * y * t5z
    sh[12] = jnp.float32(_C33) * z * (t5z - jnp.float32(2.0))
    sh[13] = jnp.float32(_C32) * x * t5z
    sh[14] = jnp.float32(_C34) * z * t8
    sh[15] = jnp.float32(_C30) * x * (xx - jnp.float32(3.0) * yy)
    return sh


@functools.lru_cache(maxsize=None)
def _make_sc_fn(N, P):
    assert P % _NW == 0
    PW = P // _NW
    B = None
    for cand in (2000, 1000, 400, 80, 16):
        if PW % cand == 0:
            B = cand
            break
    assert B is not None, PW
    NBLK = PW // B
    NG = B // _LANES

    mesh = plsc.VectorSubcoreMesh(core_axis_name="c", subcore_axis_name="s")
    f32 = jnp.float32
    i32 = jnp.int32

    @functools.partial(
        pl.kernel,
        out_type=[
            jax.ShapeDtypeStruct((P,), f32),
            jax.ShapeDtypeStruct((P * 16,), f32),
        ],
        mesh=mesh,
        compiler_params=pltpu.CompilerParams(
            needs_layout_passes=False, use_tc_tiling_on_sc=False),
        scratch_types=[
            pltpu.VMEM((B,), i32),
            pltpu.VMEM((B,), i32),
            pltpu.VMEM((B,), f32),
            pltpu.VMEM((B,), f32),
            pltpu.VMEM((B,), f32),
            pltpu.VMEM((B,), f32),
            pltpu.VMEM((B,), f32),
            pltpu.VMEM((B,), f32),
            pltpu.VMEM((B,), f32),
            pltpu.VMEM((B,), f32),
            pltpu.VMEM((B,), f32),
            pltpu.VMEM((B,), f32),
            pltpu.VMEM((B * 16,), f32),
            pltpu.VMEM((16,), f32),
            pltpu.VMEM((16,), i32),
            pltpu.SemaphoreType.DMA,
            pltpu.SemaphoreType.DMA,
        ],
    )
    def sc_fn(px_hbm, py_hbm, pz_hbm, ii_hbm, jj_hbm, sx_hbm, sy_hbm,
              sz_hbm, cell_hbm, off_hbm, r_hbm, sh_hbm, ii_v, jj_v, sx_v,
              sy_v, sz_v, xi_v, yi_v, zi_v, xj_v, yj_v, zj_v, r_v, sh_v,
              cell_v, off_v, semA, semB):
        wid = lax.axis_index("c") * 16 + lax.axis_index("s")
        base0 = wid * PW

        pltpu.sync_copy(cell_hbm, cell_v)
        pltpu.sync_copy(off_hbm, off_v)
        cv = cell_v[pl.ds(0, 16)]
        c00 = cv[0]
        c01 = cv[1]
        c02 = cv[2]
        c10 = cv[3]
        c11 = cv[4]
        c12 = cv[5]
        c20 = cv[6]
        c21 = cv[7]
        c22 = cv[8]
        off = off_v[pl.ds(0, 16)][0]

        iota = lax.broadcasted_iota(i32, (_LANES,), 0)

        def blk_body(b, carry):
            base = base0 + b * B
            pltpu.sync_copy(ii_hbm.at[pl.ds(base, B)], ii_v)
            pltpu.sync_copy(jj_hbm.at[pl.ds(base, B)], jj_v)
            pltpu.sync_copy(sx_hbm.at[pl.ds(base, B)], sx_v)
            pltpu.sync_copy(sy_hbm.at[pl.ds(base, B)], sy_v)
            pltpu.sync_copy(sz_hbm.at[pl.ds(base, B)], sz_v)

            @pl.when(off != 0)
            def _():
                def add_off(g, c):
                    sl = pl.ds(g * _LANES, _LANES)
                    ii_v[sl] = ii_v[sl] + off
                    jj_v[sl] = jj_v[sl] + off
                    return c
                lax.fori_loop(0, NG, add_off, 0)

            cp = [
                pltpu.async_copy(px_hbm.at[ii_v], xi_v, semA),
                pltpu.async_copy(py_hbm.at[ii_v], yi_v, semA),
                pltpu.async_copy(pz_hbm.at[ii_v], zi_v, semA),
                pltpu.async_copy(px_hbm.at[jj_v], xj_v, semB),
                pltpu.async_copy(py_hbm.at[jj_v], yj_v, semB),
                pltpu.async_copy(pz_hbm.at[jj_v], zj_v, semB),
            ]
            for c in cp:
                c.wait()

            def grp(g, c):
                s = g * _LANES
                sl = pl.ds(s, _LANES)
                rows16 = (s + iota) * 16
                xi = xi_v[sl]
                yi = yi_v[sl]
                zi = zi_v[sl]
                xj = xj_v[sl]
                yj = yj_v[sl]
                zj = zj_v[sl]
                svx = sx_v[sl]
                svy = sy_v[sl]
                svz = sz_v[sl]
                dx = xj - xi + svx * c00 + svy * c10 + svz * c20
                dy = yj - yi + svx * c01 + svy * c11 + svz * c21
                dz = zj - zi + svx * c02 + svy * c12 + svz * c22
                r2 = dx * dx + dy * dy + dz * dz
                rinv = _rsqrt_newton(r2)
                r_v[sl] = r2 * rinv
                ux = dx * rinv
                uy = dy * rinv
                uz = dz * rinv
                sh = _sh_components(ux, uy, uz)
                for m in range(16):
                    plsc.store_scatter(sh_v, [rows16 + m], sh[m])
                return c

            lax.fori_loop(0, NG, grp, 0)

            pltpu.sync_copy(r_v, r_hbm.at[pl.ds(base, B)])
            pltpu.sync_copy(sh_v, sh_hbm.at[pl.ds(base * 16, B * 16)])
            return carry

        lax.fori_loop(0, NBLK, blk_body, 0)

    return sc_fn


def kernel(positions, cells, species, cell_shifts, centers, pairs,
           structure_centers, structure_pairs, structure_offsets):
    N = positions.shape[0]
    P = pairs.shape[0]
    px = positions[:, 0]
    py = positions[:, 1]
    pz = positions[:, 2]
    ii = pairs[:, 0]
    jj = pairs[:, 1]
    shifts_f = cell_shifts.astype(jnp.float32)
    sx = shifts_f[:, 0]
    sy = shifts_f[:, 1]
    sz = shifts_f[:, 2]
    cell16 = jnp.pad(cells[0].reshape(-1), (0, 7))
    off16 = jnp.pad(structure_offsets, (0, 16 - structure_offsets.shape[0]))
    r, sh_flat = _make_sc_fn(N, P)(px, py, pz, ii, jj, sx, sy, sz,
                                   cell16, off16)
    return (r, sh_flat.reshape(P, 16))
